# Initial kernel scaffold; baseline (speedup 1.0000x reference)
import jax
import jax.numpy as jnp
from jax.experimental import pallas as pl


def kernel(scores):
    raise NotImplementedError("write your pallas kernel here")


# speedup vs baseline: 2.0074x; 2.0074x over previous
import jax
import jax.numpy as jnp
from jax.experimental import pallas as pl

_TOPK = 50
_ROWS_PER_BLOCK = 8


def _mask_body(x_ref, o_ref):
    v = x_ref[...]
    blk, n = v.shape
    u = jax.lax.bitcast_convert_type(v, jnp.int32)
    key = jnp.where(u >= 0, u, jnp.int32(-2147483648) - u)

    lo0 = jnp.full((blk, 1), -2147483648, jnp.int32)
    hi0 = jnp.full((blk, 1), 2147483647, jnp.int32)

    def vbody(_, carry):
        lo, hi = carry
        mid = (lo >> 1) + (hi >> 1) + (lo & hi & 1)
        cnt = jnp.sum((key >= mid).astype(jnp.int32), axis=1, keepdims=True)
        ge = cnt >= _TOPK
        return jnp.where(ge, mid, lo), jnp.where(ge, hi, mid)

    t, _ = jax.lax.fori_loop(0, 32, vbody, (lo0, hi0))

    cg = jnp.sum((key > t).astype(jnp.int32), axis=1, keepdims=True)
    rem = _TOPK - cg

    iota = jax.lax.broadcasted_iota(jnp.int32, (blk, n), 1)
    eq = key == t
    lo1 = jnp.full((blk, 1), -1, jnp.int32)
    hi1 = jnp.full((blk, 1), n - 1, jnp.int32)

    def ibody(_, carry):
        lo, hi = carry
        mid = (lo + hi + 1) >> 1
        cnt = jnp.sum((eq & (iota <= mid)).astype(jnp.int32), axis=1,
                      keepdims=True)
        ge = cnt >= rem
        return jnp.where(ge, lo, mid), jnp.where(ge, mid, hi)

    _, isel = jax.lax.fori_loop(0, 15, ibody, (lo1, hi1))

    mask = (key > t) | (eq & (iota <= isel))
    o_ref[...] = jnp.where(mask, -jnp.inf, v)


def kernel(scores):
    b, n = scores.shape
    blk = _ROWS_PER_BLOCK
    return pl.pallas_call(
        _mask_body,
        grid=(b // blk,),
        in_specs=[pl.BlockSpec((blk, n), lambda i: (i, 0))],
        out_specs=pl.BlockSpec((blk, n), lambda i: (i, 0)),
        out_shape=jax.ShapeDtypeStruct((b, n), jnp.float32),
    )(scores)


# speedup vs baseline: 4.7531x; 2.3678x over previous
import jax
import jax.numpy as jnp
from jax import lax
from jax.experimental import pallas as pl
from jax.experimental.pallas import tpu as pltpu
from jax.experimental.pallas import tpu_sc as plsc

_TOPK = 50
_N = 32768
_GROUPS = 64
_NCHUNK = _GROUPS * 16
_FAST_CHUNKS = 64
_INT_MIN = -2147483648
_INT_MAX = 2147483647


def _key16(v):
    u = lax.bitcast_convert_type(v, jnp.int32)
    return jnp.where(u >= 0, u, jnp.int32(_INT_MIN) - u)


def _splat(x):
    return jnp.full((16,), x, jnp.int32)


def _mid_safe(lo, hi):
    return (lo >> 1) + (hi >> 1) + (lo & hi & 1)


def _lane0(x):
    return lax.squeeze(lax.slice(x, (0,), (1,)), dimensions=(0,))


def _sc_body(in_hbm, out_hbm, row_v, maxk_v, candid_v, candkey_v, candidx_v,
             redbuf_v, sca):
    nc = 2
    wid = lax.axis_index("s") * nc + lax.axis_index("c")
    iota16 = lax.iota(jnp.int32, 16)
    ninf16 = jnp.full((16,), -jnp.inf, jnp.float32)
    negk16 = _splat(_INT_MIN)

    def xmax_scalar(v16):
        ks, _ = plsc.sort_key_val(v16, v16)
        return lax.squeeze(lax.slice(ks, (15,), (16,)), dimensions=(0,))

    def bsearch(lo0, hi0, count_ge, target, iters):
        sca[0] = lo0
        sca[1] = hi0

        def step(_i, _c):
            @pl.when(sca[1] - sca[0] > 1)
            def _():
                lo = sca[0]
                hi = sca[1]
                mid = _mid_safe(lo, hi)
                ge = count_ge(mid) >= target
                sca[0] = jnp.where(ge, mid, lo)
                sca[1] = jnp.where(ge, hi, mid)
            return 0

        lax.fori_loop(0, iters, step, 0)
        return sca[0]

    def row_body(j, _unused):
        r = wid * 4 + j
        pltpu.sync_copy(in_hbm.at[pl.ds(r * _N, _N)], row_v)

        def ga(g, carry):
            gmx, gmn = carry
            base = g * 512
            acc = row_v[pl.ds(base, 16)]
            for i in range(1, 32):
                acc = jnp.maximum(acc, row_v[pl.ds(base + i * 16, 16)])
            k = _key16(acc)
            maxk_v[pl.ds(g * 16, 16)] = k
            return jnp.maximum(gmx, k), jnp.minimum(gmn, k)

        gmx16, gmn16 = lax.fori_loop(0, _GROUPS, ga,
                                     (negk16, _splat(_INT_MAX)))
        gmax = xmax_scalar(gmx16)
        gminn = -xmax_scalar(-gmn16 - 1) - 1

        def cnt_maxk_ge(t):
            ts = _splat(t)
            acc = jnp.zeros((16,), jnp.int32)
            for p in range(_GROUPS):
                k = maxk_v[pl.ds(p * 16, 16)]
                acc = acc + plsc.all_reduce_population_count(k >= ts)
            return _lane0(acc)

        m50 = bsearch(gminn, gmax + 1, cnt_maxk_ge, _TOPK, 32)

        m50s = _splat(m50)

        def cb(p, off):
            k = maxk_v[pl.ds(p * 16, 16)]
            m = k >= m50s
            ids = _splat(p * 16) + iota16
            plsc.store_compressed(candid_v.at[pl.ds(off, 16)], ids, mask=m)
            return off + _lane0(plsc.all_reduce_population_count(m))

        ncand = lax.fori_loop(0, _GROUPS, cb, jnp.int32(0))
        ncs = _splat(ncand)

        def select_and_mask(npacks):
            def gc(ci, _):
                cc = candid_v[pl.ds(ci * 16, 16)]
                valid = (_splat(ci * 16) + iota16) < ncs
                ccs = jnp.where(valid, cc, 0)
                base = ((ccs >> 4) << 9) + (ccs & 15)
                for i in range(32):
                    idx = base + (i * 16)
                    k = _key16(plsc.load_gather(row_v, [idx]))
                    k = jnp.where(valid, k, negk16)
                    s = (ci * 32 + i) * 16
                    candkey_v[pl.ds(s, 16)] = k
                    candidx_v[pl.ds(s, 16)] = idx
                return 0

            lax.fori_loop(0, npacks, gc, jnp.int32(0))

            def cnt(pred):
                def body(ci, acc):
                    a = acc
                    for i in range(32):
                        s = (ci * 32 + i) * 16
                        k = candkey_v[pl.ds(s, 16)]
                        ix = candidx_v[pl.ds(s, 16)]
                        a = a + plsc.all_reduce_population_count(pred(k, ix))
                    return a

                acc = lax.fori_loop(0, npacks, body,
                                    jnp.zeros((16,), jnp.int32))
                return _lane0(acc)

            def cnt_ge(t):
                ts = _splat(t)
                return cnt(lambda k, ix: k >= ts)

            tthr = bsearch(m50, gmax + 1, cnt_ge, _TOPK, 32)

            ts = _splat(tthr)
            cg = cnt(lambda k, ix: k > ts)
            ceq = cnt(lambda k, ix: k == ts)
            rem = _TOPK - cg

            sca[2] = jnp.int32(_N - 1)

            @pl.when(cg + ceq != _TOPK)
            def _():
                def cnt_le(bound):
                    bs = _splat(bound)
                    return cnt(lambda k, ix: (k == ts) & (ix <= bs))

                sca[0] = jnp.int32(-1)
                sca[1] = jnp.int32(_N - 1)

                def istep(_i, _c):
                    @pl.when(sca[1] - sca[0] > 1)
                    def _():
                        lo = sca[0]
                        hi = sca[1]
                        mid = lo + ((hi - lo) >> 1)
                        ge = cnt_le(mid) >= rem
                        sca[0] = jnp.where(ge, lo, mid)
                        sca[1] = jnp.where(ge, mid, hi)
                    return 0

                lax.fori_loop(0, 15, istep, 0)
                sca[2] = sca[1]

            iss = _splat(sca[2])

            def scat(ci, _):
                for i in range(32):
                    s = (ci * 32 + i) * 16
                    k = candkey_v[pl.ds(s, 16)]
                    ix = candidx_v[pl.ds(s, 16)]
                    m = (k > ts) | ((k == ts) & (ix <= iss))
                    plsc.store_scatter(row_v, [ix], ninf16, mask=m)
                return 0

            lax.fori_loop(0, npacks, scat, jnp.int32(0))

        @pl.when(ncand <= _FAST_CHUNKS)
        def _():
            select_and_mask(_FAST_CHUNKS // 16)

        @pl.when(ncand > _FAST_CHUNKS)
        def _():
            select_and_mask(_NCHUNK // 16)

        pltpu.sync_copy(row_v, out_hbm.at[pl.ds(r * _N, _N)])
        return 0

    lax.fori_loop(0, 4, row_body, 0)


def kernel(scores):
    b, n = scores.shape
    flat = scores.reshape(b * n)
    mesh = plsc.VectorSubcoreMesh(core_axis_name="c", subcore_axis_name="s")
    out = pl.kernel(
        _sc_body,
        out_type=jax.ShapeDtypeStruct((b * n,), jnp.float32),
        mesh=mesh,
        compiler_params=pltpu.CompilerParams(needs_layout_passes=False),
        scratch_types=[
            pltpu.VMEM((_N,), jnp.float32),
            pltpu.VMEM((_NCHUNK,), jnp.int32),
            pltpu.VMEM((_NCHUNK + 16,), jnp.int32),
            pltpu.VMEM((_N,), jnp.int32),
            pltpu.VMEM((_N,), jnp.int32),
            pltpu.VMEM((16,), jnp.int32),
            pltpu.SMEM((4,), jnp.int32),
        ],
    )(flat)
    return out.reshape(b, n)


# speedup vs baseline: 6.9885x; 1.4703x over previous
import jax
import jax.numpy as jnp
from jax import lax
from jax.experimental import pallas as pl
from jax.experimental.pallas import tpu as pltpu
from jax.experimental.pallas import tpu_sc as plsc

_TOPK = 50
_N = 32768
_GROUPS = 64
_NCHUNK = _GROUPS * 16
_FAST_CHUNKS = 64
_INT_MIN = -2147483648
_INT_MAX = 2147483647


def _key16(v):
    u = lax.bitcast_convert_type(v, jnp.int32)
    return jnp.where(u >= 0, u, jnp.int32(_INT_MIN) - u)


def _splat(x):
    return jnp.full((16,), x, jnp.int32)


def _mid_safe(lo, hi):
    return (lo >> 1) + (hi >> 1) + (lo & hi & 1)


def _lane0(x):
    return lax.squeeze(lax.slice(x, (0,), (1,)), dimensions=(0,))


def _sc_body(in_hbm, out_hbm, row_v, maxk_v, candid_v, candkey_v, candidx_v,
             redbuf_v, sca):
    nc = 2
    wid = lax.axis_index("s") * nc + lax.axis_index("c")
    iota16 = lax.iota(jnp.int32, 16)
    ninf16 = jnp.full((16,), -jnp.inf, jnp.float32)
    negk16 = _splat(_INT_MIN)

    def xmax_scalar(v16):
        ks, _ = plsc.sort_key_val(v16, v16)
        return lax.squeeze(lax.slice(ks, (15,), (16,)), dimensions=(0,))

    def bsearch(lo0, hi0, count_ge, target, iters):
        sca[0] = lo0
        sca[1] = hi0

        def step(_i, _c):
            @pl.when(sca[1] - sca[0] > 1)
            def _():
                lo = sca[0]
                hi = sca[1]
                mid = _mid_safe(lo, hi)
                ge = count_ge(mid) >= target
                sca[0] = jnp.where(ge, mid, lo)
                sca[1] = jnp.where(ge, hi, mid)
            return 0

        lax.fori_loop(0, iters, step, 0)
        return sca[0]

    def row_body(j, _unused):
        r = wid * 4 + j
        pltpu.sync_copy(in_hbm.at[r], row_v)

        def ga(g, carry):
            gmx, gmn = carry
            base = g * 512
            acc = row_v[pl.ds(base, 16)]
            for i in range(1, 32):
                acc = jnp.maximum(acc, row_v[pl.ds(base + i * 16, 16)])
            k = _key16(acc)
            maxk_v[pl.ds(g * 16, 16)] = k
            return jnp.maximum(gmx, k), jnp.minimum(gmn, k)

        gmx16, gmn16 = lax.fori_loop(0, _GROUPS, ga,
                                     (negk16, _splat(_INT_MAX)))
        gmax = xmax_scalar(gmx16)
        gminn = -xmax_scalar(-gmn16 - 1) - 1

        def cnt_maxk_ge(t):
            ts = _splat(t)
            acc = jnp.zeros((16,), jnp.int32)
            for p in range(_GROUPS):
                k = maxk_v[pl.ds(p * 16, 16)]
                acc = acc + plsc.all_reduce_population_count(k >= ts)
            return _lane0(acc)

        m50 = bsearch(gminn, gmax + 1, cnt_maxk_ge, _TOPK, 32)

        m50s = _splat(m50)

        def cb(p, off):
            k = maxk_v[pl.ds(p * 16, 16)]
            m = k >= m50s
            ids = _splat(p * 16) + iota16
            plsc.store_compressed(candid_v.at[pl.ds(off, 16)], ids, mask=m)
            return off + _lane0(plsc.all_reduce_population_count(m))

        ncand = lax.fori_loop(0, _GROUPS, cb, jnp.int32(0))
        ncs = _splat(ncand)

        def select_and_mask(npacks):
            def gc(ci, _):
                cc = candid_v[pl.ds(ci * 16, 16)]
                valid = (_splat(ci * 16) + iota16) < ncs
                ccs = jnp.where(valid, cc, 0)
                base = ((ccs >> 4) << 9) + (ccs & 15)
                for i in range(32):
                    idx = base + (i * 16)
                    k = _key16(plsc.load_gather(row_v, [idx]))
                    k = jnp.where(valid, k, negk16)
                    s = (ci * 32 + i) * 16
                    candkey_v[pl.ds(s, 16)] = k
                    candidx_v[pl.ds(s, 16)] = idx
                return 0

            lax.fori_loop(0, npacks, gc, jnp.int32(0))

            def cnt(pred):
                def body(ci, acc):
                    a = acc
                    for i in range(32):
                        s = (ci * 32 + i) * 16
                        k = candkey_v[pl.ds(s, 16)]
                        ix = candidx_v[pl.ds(s, 16)]
                        a = a + plsc.all_reduce_population_count(pred(k, ix))
                    return a

                acc = lax.fori_loop(0, npacks, body,
                                    jnp.zeros((16,), jnp.int32))
                return _lane0(acc)

            def cnt_ge(t):
                ts = _splat(t)
                return cnt(lambda k, ix: k >= ts)

            tthr = bsearch(m50, gmax + 1, cnt_ge, _TOPK, 32)

            ts = _splat(tthr)
            cg = cnt(lambda k, ix: k > ts)
            ceq = cnt(lambda k, ix: k == ts)
            rem = _TOPK - cg

            sca[2] = jnp.int32(_N - 1)

            @pl.when(cg + ceq != _TOPK)
            def _():
                def cnt_le(bound):
                    bs = _splat(bound)
                    return cnt(lambda k, ix: (k == ts) & (ix <= bs))

                sca[0] = jnp.int32(-1)
                sca[1] = jnp.int32(_N - 1)

                def istep(_i, _c):
                    @pl.when(sca[1] - sca[0] > 1)
                    def _():
                        lo = sca[0]
                        hi = sca[1]
                        mid = lo + ((hi - lo) >> 1)
                        ge = cnt_le(mid) >= rem
                        sca[0] = jnp.where(ge, lo, mid)
                        sca[1] = jnp.where(ge, mid, hi)
                    return 0

                lax.fori_loop(0, 15, istep, 0)
                sca[2] = sca[1]

            iss = _splat(sca[2])

            def scat(ci, _):
                for i in range(32):
                    s = (ci * 32 + i) * 16
                    k = candkey_v[pl.ds(s, 16)]
                    ix = candidx_v[pl.ds(s, 16)]
                    m = (k > ts) | ((k == ts) & (ix <= iss))
                    plsc.store_scatter(row_v, [ix], ninf16, mask=m)
                return 0

            lax.fori_loop(0, npacks, scat, jnp.int32(0))

        @pl.when(ncand <= _FAST_CHUNKS)
        def _():
            select_and_mask(_FAST_CHUNKS // 16)

        @pl.when(ncand > _FAST_CHUNKS)
        def _():
            select_and_mask(_NCHUNK // 16)

        pltpu.sync_copy(row_v, out_hbm.at[r])
        return 0

    lax.fori_loop(0, 4, row_body, 0)


def kernel(scores):
    b, n = scores.shape
    mesh = plsc.VectorSubcoreMesh(core_axis_name="c", subcore_axis_name="s")
    return pl.kernel(
        _sc_body,
        out_type=jax.ShapeDtypeStruct((b, n), jnp.float32),
        mesh=mesh,
        compiler_params=pltpu.CompilerParams(needs_layout_passes=False),
        scratch_types=[
            pltpu.VMEM((_N,), jnp.float32),
            pltpu.VMEM((_NCHUNK,), jnp.int32),
            pltpu.VMEM((_NCHUNK + 16,), jnp.int32),
            pltpu.VMEM((_N,), jnp.int32),
            pltpu.VMEM((_N,), jnp.int32),
            pltpu.VMEM((16,), jnp.int32),
            pltpu.SMEM((4,), jnp.int32),
        ],
    )(scores)


# speedup vs baseline: 8.1010x; 1.1592x over previous
import jax
import jax.numpy as jnp
from jax import lax
from jax.experimental import pallas as pl
from jax.experimental.pallas import tpu as pltpu
from jax.experimental.pallas import tpu_sc as plsc

_TOPK = 50
_N = 32768
_GROUPS = 64
_NCHUNK = _GROUPS * 16
_FAST_CHUNKS = 64
_INT_MIN = -2147483648
_INT_MAX = 2147483647


def _key16(v):
    u = lax.bitcast_convert_type(v, jnp.int32)
    return jnp.where(u >= 0, u, jnp.int32(_INT_MIN) - u)


def _splat(x):
    return jnp.full((16,), x, jnp.int32)


def _mid_safe(lo, hi):
    return (lo >> 1) + (hi >> 1) + (lo & hi & 1)


def _lane0(x):
    return lax.squeeze(lax.slice(x, (0,), (1,)), dimensions=(0,))


def _sc_body(in_hbm, out_hbm, bufa_v, bufb_v, maxk_v, candid_v, candkey_v,
             sca, sem_ia, sem_ib, sem_oa, sem_ob):
    nc = 2
    wid = lax.axis_index("s") * nc + lax.axis_index("c")
    iota16 = lax.iota(jnp.int32, 16)
    ninf16 = jnp.full((16,), -jnp.inf, jnp.float32)
    negk16 = _splat(_INT_MIN)
    bufs = (bufa_v, bufb_v)
    sem_in = (sem_ia, sem_ib)
    sem_out = (sem_oa, sem_ob)

    def xmax_scalar(v16):
        ks, _ = plsc.sort_key_val(v16, v16)
        return lax.squeeze(lax.slice(ks, (15,), (16,)), dimensions=(0,))

    def bsearch(lo0, hi0, count_ge, target, iters):
        sca[0] = lo0
        sca[1] = hi0

        def step(_i, _c):
            @pl.when(sca[1] - sca[0] > 1)
            def _():
                lo = sca[0]
                hi = sca[1]
                mid = _mid_safe(lo, hi)
                ge = count_ge(mid) >= target
                sca[0] = jnp.where(ge, mid, lo)
                sca[1] = jnp.where(ge, hi, mid)
            return 0

        lax.fori_loop(0, iters, step, 0)
        return sca[0]

    def chunk_stats(buf):
        def ga(g, carry):
            gmx, gmn = carry
            base = g * 512
            acc = buf[pl.ds(base, 16)]
            for i in range(1, 32):
                acc = jnp.maximum(acc, buf[pl.ds(base + i * 16, 16)])
            k = _key16(acc)
            maxk_v[pl.ds(g * 16, 16)] = k
            return jnp.maximum(gmx, k), jnp.minimum(gmn, k)

        gmx16, gmn16 = lax.fori_loop(0, _GROUPS, ga,
                                     (negk16, _splat(_INT_MAX)))
        gmax = xmax_scalar(gmx16)
        gminn = -xmax_scalar(-gmn16 - 1) - 1
        return gmax, gminn

    def cnt_maxk_ge(t):
        ts = _splat(t)
        acc = jnp.zeros((16,), jnp.int32)
        for p in range(_GROUPS):
            k = maxk_v[pl.ds(p * 16, 16)]
            acc = acc + plsc.all_reduce_population_count(k >= ts)
        return _lane0(acc)

    def compact_candidates(m50):
        m50s = _splat(m50)

        def cb(p, off):
            k = maxk_v[pl.ds(p * 16, 16)]
            m = k >= m50s
            ids = _splat(p * 16) + iota16
            plsc.store_compressed(candid_v.at[pl.ds(off, 16)], ids, mask=m)
            return off + _lane0(plsc.all_reduce_population_count(m))

        return lax.fori_loop(0, _GROUPS, cb, jnp.int32(0))

    npacks = _FAST_CHUNKS // 16

    def cand_pack(ci, ncs):
        cc = candid_v[pl.ds(ci * 16, 16)]
        valid = (_splat(ci * 16) + iota16) < ncs
        ccs = jnp.where(valid, cc, 0)
        base = ((ccs >> 4) << 9) + (ccs & 15)
        return base, valid

    def fast_row(j, buf, r):
        gmax, gminn = chunk_stats(buf)
        m50 = bsearch(gminn, gmax + 1, cnt_maxk_ge, _TOPK, 32)
        ncand = compact_candidates(m50)
        ncs = _splat(ncand)

        @pl.when(ncand <= _FAST_CHUNKS)
        def _():
            def gc(ci, _):
                base, valid = cand_pack(ci, ncs)
                for i in range(32):
                    k = _key16(plsc.load_gather(buf, [base + i * 16]))
                    k = jnp.where(valid, k, negk16)
                    candkey_v[pl.ds((ci * 32 + i) * 16, 16)] = k
                return 0

            lax.fori_loop(0, npacks, gc, jnp.int32(0))

            def cnt_ge(t):
                ts = _splat(t)
                acc = jnp.zeros((16,), jnp.int32)
                for s in range(npacks * 32):
                    k = candkey_v[pl.ds(s * 16, 16)]
                    acc = acc + plsc.all_reduce_population_count(k >= ts)
                return _lane0(acc)

            tthr = bsearch(m50, gmax + 1, cnt_ge, _TOPK, 32)
            exact = cnt_ge(tthr) == _TOPK
            sca[3 + j] = jnp.where(exact, 0, 1)

            @pl.when(exact)
            def _():
                ts = _splat(tthr)

                def scat(ci, _):
                    base, _valid = cand_pack(ci, ncs)
                    for i in range(32):
                        s = (ci * 32 + i) * 16
                        k = candkey_v[pl.ds(s, 16)]
                        plsc.store_scatter(buf, [base + i * 16], ninf16,
                                           mask=k >= ts)
                    return 0

                lax.fori_loop(0, npacks, scat, jnp.int32(0))

        @pl.when(ncand > _FAST_CHUNKS)
        def _():
            sca[3 + j] = jnp.int32(1)

    def cnt_row_mode(mode, t, bound):
        ts = _splat(t)
        bs = _splat(bound)
        m0 = _splat(mode) == 0
        m1 = _splat(mode) == 1

        def body(g, acc):
            a = acc
            base = g * 512
            for i in range(32):
                k = _key16(bufa_v[pl.ds(base + i * 16, 16)])
                idx = _splat(base + i * 16) + iota16
                m = jnp.where(m0, k >= ts,
                              jnp.where(m1, k > ts,
                                        (k == ts) & (idx <= bs)))
                a = a + plsc.all_reduce_population_count(m)
            return a

        return _lane0(lax.fori_loop(0, _GROUPS, body,
                                    jnp.zeros((16,), jnp.int32)))

    def cleanup_row(r):
        pltpu.sync_copy(in_hbm.at[r], bufa_v)
        tthr = bsearch(jnp.int32(_INT_MIN), jnp.int32(_INT_MAX),
                       lambda t: cnt_row_mode(jnp.int32(0), t, jnp.int32(0)),
                       _TOPK, 32)
        cg = cnt_row_mode(jnp.int32(1), tthr, jnp.int32(0))
        rem = _TOPK - cg

        sca[0] = jnp.int32(-1)
        sca[1] = jnp.int32(_N - 1)

        def istep(_i, _c):
            @pl.when(sca[1] - sca[0] > 1)
            def _():
                lo = sca[0]
                hi = sca[1]
                mid = lo + ((hi - lo) >> 1)
                ge = cnt_row_mode(jnp.int32(2), tthr, mid) >= rem
                sca[0] = jnp.where(ge, lo, mid)
                sca[1] = jnp.where(ge, mid, hi)
            return 0

        lax.fori_loop(0, 15, istep, 0)
        isel = sca[1]

        ts = _splat(tthr)
        iss = _splat(isel)

        def rw(g, _):
            base = g * 512
            for i in range(32):
                sl = pl.ds(base + i * 16, 16)
                v = bufa_v[sl]
                k = _key16(v)
                idx = _splat(base + i * 16) + iota16
                m = (k > ts) | ((k == ts) & (idx <= iss))
                bufa_v[sl] = jnp.where(m, ninf16, v)
            return 0

        lax.fori_loop(0, _GROUPS, rw, 0)
        pltpu.sync_copy(bufa_v, out_hbm.at[r])

    in_desc = [None, None]
    out_desc = [None, None]
    in_desc[0] = pltpu.async_copy(in_hbm.at[wid * 4], bufs[0], sem_in[0])
    for j in range(4):
        p = j & 1
        r = wid * 4 + j
        in_desc[p].wait()
        if j >= 1:
            out_desc[1 - p].wait()
        if j < 3:
            in_desc[1 - p] = pltpu.async_copy(in_hbm.at[r + 1], bufs[1 - p],
                                              sem_in[1 - p])
        fast_row(j, bufs[p], r)
        out_desc[p] = pltpu.async_copy(bufs[p], out_hbm.at[r], sem_out[p])
    out_desc[1].wait()

    def cl(j, _):
        @pl.when(sca[3 + j] == 1)
        def _():
            cleanup_row(wid * 4 + j)
        return 0

    lax.fori_loop(0, 4, cl, 0)


def kernel(scores):
    b, n = scores.shape
    mesh = plsc.VectorSubcoreMesh(core_axis_name="c", subcore_axis_name="s")
    return pl.kernel(
        _sc_body,
        out_type=jax.ShapeDtypeStruct((b, n), jnp.float32),
        mesh=mesh,
        compiler_params=pltpu.CompilerParams(needs_layout_passes=False),
        scratch_types=[
            pltpu.VMEM((_N,), jnp.float32),
            pltpu.VMEM((_N,), jnp.float32),
            pltpu.VMEM((_NCHUNK,), jnp.int32),
            pltpu.VMEM((_NCHUNK + 16,), jnp.int32),
            pltpu.VMEM((_FAST_CHUNKS * 32,), jnp.int32),
            pltpu.SMEM((8,), jnp.int32),
            pltpu.SemaphoreType.DMA,
            pltpu.SemaphoreType.DMA,
            pltpu.SemaphoreType.DMA,
            pltpu.SemaphoreType.DMA,
        ],
    )(scores)


# speedup vs baseline: 8.1019x; 1.0001x over previous
import jax
import jax.numpy as jnp
from jax import lax
from jax.experimental import pallas as pl
from jax.experimental.pallas import tpu as pltpu
from jax.experimental.pallas import tpu_sc as plsc

_TOPK = 50
_N = 32768
_GROUPS = 64
_NCHUNK = _GROUPS * 16
_FAST_CHUNKS = 64
_INT_MIN = -2147483648
_INT_MAX = 2147483647


def _key16(v):
    u = lax.bitcast_convert_type(v, jnp.int32)
    return jnp.where(u >= 0, u, jnp.int32(_INT_MIN) - u)


def _splat(x):
    return jnp.full((16,), x, jnp.int32)


def _mid_safe(lo, hi):
    return (lo >> 1) + (hi >> 1) + (lo & hi & 1)


def _lane0(x):
    return lax.squeeze(lax.slice(x, (0,), (1,)), dimensions=(0,))


def _sc_body(in_hbm, out_hbm, bufa_v, bufb_v, maxk_v, candid_v, candkey_v,
             sca, sem_ia, sem_ib, sem_oa, sem_ob):
    nc = 2
    wid = lax.axis_index("s") * nc + lax.axis_index("c")
    iota16 = lax.iota(jnp.int32, 16)
    ninf16 = jnp.full((16,), -jnp.inf, jnp.float32)
    negk16 = _splat(_INT_MIN)
    bufs = (bufa_v, bufb_v)
    sem_in = (sem_ia, sem_ib)
    sem_out = (sem_oa, sem_ob)

    def xmax_scalar(v16):
        ks, _ = plsc.sort_key_val(v16, v16)
        return lax.squeeze(lax.slice(ks, (15,), (16,)), dimensions=(0,))

    def bsearch(lo0, hi0, count_ge, target, iters):
        sca[0] = lo0
        sca[1] = hi0

        def step(_i, _c):
            @pl.when(sca[1] - sca[0] > 1)
            def _():
                lo = sca[0]
                hi = sca[1]
                mid = _mid_safe(lo, hi)
                ge = count_ge(mid) >= target
                sca[0] = jnp.where(ge, mid, lo)
                sca[1] = jnp.where(ge, hi, mid)
            return 0

        lax.fori_loop(0, iters, step, 0)
        return sca[0]

    def chunk_stats(buf):
        def ga(g, carry):
            gmx, gmn = carry
            base = g * 512
            acc = buf[pl.ds(base, 16)]
            for i in range(1, 32):
                acc = jnp.maximum(acc, buf[pl.ds(base + i * 16, 16)])
            k = _key16(acc)
            maxk_v[pl.ds(g * 16, 16)] = k
            return jnp.maximum(gmx, k), jnp.minimum(gmn, k)

        gmx16, gmn16 = lax.fori_loop(0, _GROUPS, ga,
                                     (negk16, _splat(_INT_MAX)))
        gmax = xmax_scalar(gmx16)
        gminn = -xmax_scalar(-gmn16 - 1) - 1
        return gmax, gminn

    def cnt_maxk_ge(t):
        ts = _splat(t)
        acc = jnp.zeros((16,), jnp.int32)
        for p in range(_GROUPS):
            k = maxk_v[pl.ds(p * 16, 16)]
            acc = acc + plsc.all_reduce_population_count(k >= ts)
        return _lane0(acc)

    def compact_candidates(m50):
        m50s = _splat(m50)

        def cb(p, off):
            k = maxk_v[pl.ds(p * 16, 16)]
            m = k >= m50s
            ids = _splat(p * 16) + iota16
            plsc.store_compressed(candid_v.at[pl.ds(off, 16)], ids, mask=m)
            return off + _lane0(plsc.all_reduce_population_count(m))

        return lax.fori_loop(0, _GROUPS, cb, jnp.int32(0))

    npacks = _FAST_CHUNKS // 16

    def cand_pack(ci, ncs):
        cc = candid_v[pl.ds(ci * 16, 16)]
        valid = (_splat(ci * 16) + iota16) < ncs
        ccs = jnp.where(valid, cc, 0)
        base = ((ccs >> 4) << 9) + (ccs & 15)
        return base, valid

    def fast_row(j, buf, r):
        gmax, gminn = chunk_stats(buf)
        m50 = bsearch(gminn, gmax + 1, cnt_maxk_ge, _TOPK, 32)
        ncand = compact_candidates(m50)
        ncs = _splat(ncand)

        @pl.when(ncand <= _FAST_CHUNKS)
        def _():
            def gc(ci, _):
                base, valid = cand_pack(ci, ncs)
                for i in range(32):
                    k = _key16(plsc.load_gather(buf, [base + i * 16]))
                    k = jnp.where(valid, k, negk16)
                    candkey_v[pl.ds((ci * 32 + i) * 16, 16)] = k
                return 0

            lax.fori_loop(0, npacks, gc, jnp.int32(0))

            def cnt_ge(t):
                ts = _splat(t)
                acc = jnp.zeros((16,), jnp.int32)
                for s in range(npacks * 32):
                    k = candkey_v[pl.ds(s * 16, 16)]
                    acc = acc + plsc.all_reduce_population_count(k >= ts)
                return _lane0(acc)

            tthr = bsearch(m50, gmax + 1, cnt_ge, _TOPK, 32)
            exact = cnt_ge(tthr) == _TOPK
            sca[3 + j] = jnp.where(exact, 0, 1)

            @pl.when(exact)
            def _():
                ts = _splat(tthr)

                def scat(ci, _):
                    base, _valid = cand_pack(ci, ncs)
                    for i in range(32):
                        s = (ci * 32 + i) * 16
                        k = candkey_v[pl.ds(s, 16)]
                        plsc.store_scatter(buf, [base + i * 16], ninf16,
                                           mask=k >= ts)
                    return 0

                lax.fori_loop(0, npacks, scat, jnp.int32(0))

        @pl.when(ncand > _FAST_CHUNKS)
        def _():
            sca[3 + j] = jnp.int32(1)

    def cnt_row_mode(mode, t, bound):
        ts = _splat(t)
        bs = _splat(bound)
        m0 = _splat(mode) == 0
        m1 = _splat(mode) == 1

        def body(g, acc):
            a = acc
            base = g * 512
            for i in range(32):
                k = _key16(bufa_v[pl.ds(base + i * 16, 16)])
                idx = _splat(base + i * 16) + iota16
                m = jnp.where(m0, k >= ts,
                              jnp.where(m1, k > ts,
                                        (k == ts) & (idx <= bs)))
                a = a + plsc.all_reduce_population_count(m)
            return a

        return _lane0(lax.fori_loop(0, _GROUPS, body,
                                    jnp.zeros((16,), jnp.int32)))

    def cleanup_row(r):
        pltpu.sync_copy(in_hbm.at[r], bufa_v)
        tthr = bsearch(jnp.int32(_INT_MIN), jnp.int32(_INT_MAX),
                       lambda t: cnt_row_mode(jnp.int32(0), t, jnp.int32(0)),
                       _TOPK, 32)
        cg = cnt_row_mode(jnp.int32(1), tthr, jnp.int32(0))
        rem = _TOPK - cg

        sca[0] = jnp.int32(-1)
        sca[1] = jnp.int32(_N - 1)

        def istep(_i, _c):
            @pl.when(sca[1] - sca[0] > 1)
            def _():
                lo = sca[0]
                hi = sca[1]
                mid = lo + ((hi - lo) >> 1)
                ge = cnt_row_mode(jnp.int32(2), tthr, mid) >= rem
                sca[0] = jnp.where(ge, lo, mid)
                sca[1] = jnp.where(ge, mid, hi)
            return 0

        lax.fori_loop(0, 15, istep, 0)
        isel = sca[1]

        ts = _splat(tthr)
        iss = _splat(isel)

        def rw(g, _):
            base = g * 512
            for i in range(32):
                sl = pl.ds(base + i * 16, 16)
                v = bufa_v[sl]
                k = _key16(v)
                idx = _splat(base + i * 16) + iota16
                m = (k > ts) | ((k == ts) & (idx <= iss))
                bufa_v[sl] = jnp.where(m, ninf16, v)
            return 0

        lax.fori_loop(0, _GROUPS, rw, 0)
        pltpu.sync_copy(bufa_v, out_hbm.at[r])

    in_desc = [None, None]
    out_desc = [None, None]
    in_desc[0] = pltpu.async_copy(in_hbm.at[wid * 4], bufs[0], sem_in[0])
    for j in range(4):
        p = j & 1
        r = wid * 4 + j
        in_desc[p].wait()
        if j >= 1:
            out_desc[1 - p].wait()
        if j < 3:
            in_desc[1 - p] = pltpu.async_copy(in_hbm.at[r + 1], bufs[1 - p],
                                              sem_in[1 - p])
        fast_row(j, bufs[p], r)
        out_desc[p] = pltpu.async_copy(bufs[p], out_hbm.at[r], sem_out[p])
    out_desc[1].wait()

    def cl(j, _):
        @pl.when(sca[3 + j] == 1)
        def _():
            cleanup_row(wid * 4 + j)
        return 0

    lax.fori_loop(0, 4, cl, 0)


def kernel(scores):
    b, n = scores.shape
    mesh = plsc.VectorSubcoreMesh(core_axis_name="c", subcore_axis_name="s")
    return pl.kernel(
        _sc_body,
        out_type=jax.ShapeDtypeStruct((b, n), jnp.float32),
        mesh=mesh,
        compiler_params=pltpu.CompilerParams(needs_layout_passes=False,
                                             skip_device_barrier=True),
        scratch_types=[
            pltpu.VMEM((_N,), jnp.float32),
            pltpu.VMEM((_N,), jnp.float32),
            pltpu.VMEM((_NCHUNK,), jnp.int32),
            pltpu.VMEM((_NCHUNK + 16,), jnp.int32),
            pltpu.VMEM((_FAST_CHUNKS * 32,), jnp.int32),
            pltpu.SMEM((8,), jnp.int32),
            pltpu.SemaphoreType.DMA,
            pltpu.SemaphoreType.DMA,
            pltpu.SemaphoreType.DMA,
            pltpu.SemaphoreType.DMA,
        ],
    )(scores)


# speedup vs baseline: 8.5169x; 1.0512x over previous
import jax
import jax.numpy as jnp
from jax import lax
from jax.experimental import pallas as pl
from jax.experimental.pallas import tpu as pltpu
from jax.experimental.pallas import tpu_sc as plsc

_TOPK = 50
_N = 32768
_GROUPS = 64
_NCHUNK = _GROUPS * 16
_FAST_CHUNKS = 64
_INT_MIN = -2147483648
_INT_MAX = 2147483647


def _key16(v):
    u = lax.bitcast_convert_type(v, jnp.int32)
    return jnp.where(u >= 0, u, jnp.int32(_INT_MIN) - u)


def _splat(x):
    return jnp.full((16,), x, jnp.int32)


def _mid_safe(lo, hi):
    return (lo >> 1) + (hi >> 1) + (lo & hi & 1)


def _lane0(x):
    return lax.squeeze(lax.slice(x, (0,), (1,)), dimensions=(0,))


def _sc_body(in_hbm, out_hbm, bufa_v, bufb_v, maxk_v, candid_v, candkey_v,
             sca, sem_ia, sem_ib, sem_oa, sem_ob):
    nc = 2
    wid = lax.axis_index("s") * nc + lax.axis_index("c")
    iota16 = lax.iota(jnp.int32, 16)
    ninf16 = jnp.full((16,), -jnp.inf, jnp.float32)
    negk16 = _splat(_INT_MIN)
    bufs = (bufa_v, bufb_v)
    sem_in = (sem_ia, sem_ib)
    sem_out = (sem_oa, sem_ob)

    def xmax_scalar(v16):
        ks, _ = plsc.sort_key_val(v16, v16)
        return lax.squeeze(lax.slice(ks, (15,), (16,)), dimensions=(0,))

    def bsearch(lo0, hi0, count_ge, target, iters):
        sca[0] = lo0
        sca[1] = hi0

        def step(_i, _c):
            @pl.when(sca[1] - sca[0] > 1)
            def _():
                lo = sca[0]
                hi = sca[1]
                mid = _mid_safe(lo, hi)
                ge = count_ge(mid) >= target
                sca[0] = jnp.where(ge, mid, lo)
                sca[1] = jnp.where(ge, hi, mid)
            return 0

        lax.fori_loop(0, iters, step, 0)
        return sca[0]

    def bsearch_window(lo0, hi0, count_ge, target, cap, iters):
        sca[0] = lo0
        sca[1] = hi0
        sca[2] = jnp.int32(0)

        def step(_i, _c):
            @pl.when((sca[2] == 0) & (sca[1] - sca[0] > 1))
            def _():
                lo = sca[0]
                hi = sca[1]
                mid = _mid_safe(lo, hi)
                c = count_ge(mid)
                ge = c >= target
                sca[0] = jnp.where(ge, mid, lo)
                sca[1] = jnp.where(ge, hi, mid)
                sca[2] = jnp.where(ge & (c <= cap), 1, 0)
            return 0

        lax.fori_loop(0, iters, step, 0)
        return sca[0]

    def chunk_stats(buf):
        def ga(g, carry):
            gmx, gmn = carry
            base = g * 512
            acc = buf[pl.ds(base, 16)]
            for i in range(1, 32):
                acc = jnp.maximum(acc, buf[pl.ds(base + i * 16, 16)])
            k = _key16(acc)
            maxk_v[pl.ds(g * 16, 16)] = k
            return jnp.maximum(gmx, k), jnp.minimum(gmn, k)

        gmx16, gmn16 = lax.fori_loop(0, _GROUPS, ga,
                                     (negk16, _splat(_INT_MAX)))
        gmax = xmax_scalar(gmx16)
        gminn = -xmax_scalar(-gmn16 - 1) - 1
        return gmax, gminn

    def cnt_maxk_ge(t):
        ts = _splat(t)
        acc = jnp.zeros((16,), jnp.int32)
        for p in range(_GROUPS):
            k = maxk_v[pl.ds(p * 16, 16)]
            acc = acc + plsc.all_reduce_population_count(k >= ts)
        return _lane0(acc)

    def compact_candidates(m50):
        m50s = _splat(m50)

        def cb(p, off):
            k = maxk_v[pl.ds(p * 16, 16)]
            m = k >= m50s
            ids = _splat(p * 16) + iota16
            plsc.store_compressed(candid_v.at[pl.ds(off, 16)], ids, mask=m)
            return off + _lane0(plsc.all_reduce_population_count(m))

        return lax.fori_loop(0, _GROUPS, cb, jnp.int32(0))

    npacks = _FAST_CHUNKS // 16

    def cand_pack(ci, ncs):
        cc = candid_v[pl.ds(ci * 16, 16)]
        valid = (_splat(ci * 16) + iota16) < ncs
        ccs = jnp.where(valid, cc, 0)
        base = ((ccs >> 4) << 9) + (ccs & 15)
        return base, valid

    def fast_row(j, buf, r):
        gmax, gminn = chunk_stats(buf)
        m50 = bsearch_window(gminn, gmax + 1, cnt_maxk_ge, _TOPK,
                             _FAST_CHUNKS, 32)
        ncand = compact_candidates(m50)
        ncs = _splat(ncand)

        @pl.when(ncand <= _FAST_CHUNKS)
        def _():
            def gc(ci, _):
                base, valid = cand_pack(ci, ncs)
                for i in range(32):
                    k = _key16(plsc.load_gather(buf, [base + i * 16]))
                    k = jnp.where(valid, k, negk16)
                    candkey_v[pl.ds((ci * 32 + i) * 16, 16)] = k
                return 0

            lax.fori_loop(0, npacks, gc, jnp.int32(0))

            def cnt_ge(t):
                ts = _splat(t)
                acc = jnp.zeros((16,), jnp.int32)
                for s in range(npacks * 32):
                    k = candkey_v[pl.ds(s * 16, 16)]
                    acc = acc + plsc.all_reduce_population_count(k >= ts)
                return _lane0(acc)

            tthr = bsearch(m50, gmax + 1, cnt_ge, _TOPK, 32)
            exact = cnt_ge(tthr) == _TOPK
            sca[3 + j] = jnp.where(exact, 0, 1)

            @pl.when(exact)
            def _():
                ts = _splat(tthr)

                def scat(ci, _):
                    base, _valid = cand_pack(ci, ncs)
                    for i in range(32):
                        s = (ci * 32 + i) * 16
                        k = candkey_v[pl.ds(s, 16)]
                        plsc.store_scatter(buf, [base + i * 16], ninf16,
                                           mask=k >= ts)
                    return 0

                lax.fori_loop(0, npacks, scat, jnp.int32(0))

        @pl.when(ncand > _FAST_CHUNKS)
        def _():
            sca[3 + j] = jnp.int32(1)

    def cnt_row_mode(mode, t, bound):
        ts = _splat(t)
        bs = _splat(bound)
        m0 = _splat(mode) == 0
        m1 = _splat(mode) == 1

        def body(g, acc):
            a = acc
            base = g * 512
            for i in range(32):
                k = _key16(bufa_v[pl.ds(base + i * 16, 16)])
                idx = _splat(base + i * 16) + iota16
                m = jnp.where(m0, k >= ts,
                              jnp.where(m1, k > ts,
                                        (k == ts) & (idx <= bs)))
                a = a + plsc.all_reduce_population_count(m)
            return a

        return _lane0(lax.fori_loop(0, _GROUPS, body,
                                    jnp.zeros((16,), jnp.int32)))

    def cleanup_row(r):
        pltpu.sync_copy(in_hbm.at[r], bufa_v)
        tthr = bsearch(jnp.int32(_INT_MIN), jnp.int32(_INT_MAX),
                       lambda t: cnt_row_mode(jnp.int32(0), t, jnp.int32(0)),
                       _TOPK, 32)
        cg = cnt_row_mode(jnp.int32(1), tthr, jnp.int32(0))
        rem = _TOPK - cg

        sca[0] = jnp.int32(-1)
        sca[1] = jnp.int32(_N - 1)

        def istep(_i, _c):
            @pl.when(sca[1] - sca[0] > 1)
            def _():
                lo = sca[0]
                hi = sca[1]
                mid = lo + ((hi - lo) >> 1)
                ge = cnt_row_mode(jnp.int32(2), tthr, mid) >= rem
                sca[0] = jnp.where(ge, lo, mid)
                sca[1] = jnp.where(ge, mid, hi)
            return 0

        lax.fori_loop(0, 15, istep, 0)
        isel = sca[1]

        ts = _splat(tthr)
        iss = _splat(isel)

        def rw(g, _):
            base = g * 512
            for i in range(32):
                sl = pl.ds(base + i * 16, 16)
                v = bufa_v[sl]
                k = _key16(v)
                idx = _splat(base + i * 16) + iota16
                m = (k > ts) | ((k == ts) & (idx <= iss))
                bufa_v[sl] = jnp.where(m, ninf16, v)
            return 0

        lax.fori_loop(0, _GROUPS, rw, 0)
        pltpu.sync_copy(bufa_v, out_hbm.at[r])

    in_desc = [None, None]
    out_desc = [None, None]
    in_desc[0] = pltpu.async_copy(in_hbm.at[wid * 4], bufs[0], sem_in[0])
    for j in range(4):
        p = j & 1
        r = wid * 4 + j
        in_desc[p].wait()
        if j >= 1:
            out_desc[1 - p].wait()
        if j < 3:
            in_desc[1 - p] = pltpu.async_copy(in_hbm.at[r + 1], bufs[1 - p],
                                              sem_in[1 - p])
        fast_row(j, bufs[p], r)
        out_desc[p] = pltpu.async_copy(bufs[p], out_hbm.at[r], sem_out[p])
    out_desc[1].wait()

    def cl(j, _):
        @pl.when(sca[3 + j] == 1)
        def _():
            cleanup_row(wid * 4 + j)
        return 0

    lax.fori_loop(0, 4, cl, 0)


def kernel(scores):
    b, n = scores.shape
    mesh = plsc.VectorSubcoreMesh(core_axis_name="c", subcore_axis_name="s")
    return pl.kernel(
        _sc_body,
        out_type=jax.ShapeDtypeStruct((b, n), jnp.float32),
        mesh=mesh,
        compiler_params=pltpu.CompilerParams(needs_layout_passes=False),
        scratch_types=[
            pltpu.VMEM((_N,), jnp.float32),
            pltpu.VMEM((_N,), jnp.float32),
            pltpu.VMEM((_NCHUNK,), jnp.int32),
            pltpu.VMEM((_NCHUNK + 16,), jnp.int32),
            pltpu.VMEM((_FAST_CHUNKS * 32,), jnp.int32),
            pltpu.SMEM((8,), jnp.int32),
            pltpu.SemaphoreType.DMA,
            pltpu.SemaphoreType.DMA,
            pltpu.SemaphoreType.DMA,
            pltpu.SemaphoreType.DMA,
        ],
    )(scores)


# speedup vs baseline: 8.6079x; 1.0107x over previous
import jax
import jax.numpy as jnp
from jax import lax
from jax.experimental import pallas as pl
from jax.experimental.pallas import tpu as pltpu
from jax.experimental.pallas import tpu_sc as plsc

_TOPK = 50
_N = 32768
_GROUPS = 64
_NCHUNK = _GROUPS * 16
_FAST_CHUNKS = 64
_INT_MIN = -2147483648
_INT_MAX = 2147483647


def _key16(v):
    u = lax.bitcast_convert_type(v, jnp.int32)
    return jnp.where(u >= 0, u, jnp.int32(_INT_MIN) - u)


def _splat(x):
    return jnp.full((16,), x, jnp.int32)


def _mid_safe(lo, hi):
    return (lo >> 1) + (hi >> 1) + (lo & hi & 1)


def _lane0(x):
    return lax.squeeze(lax.slice(x, (0,), (1,)), dimensions=(0,))


def _sc_body(in_hbm, out_hbm, bufa_v, bufb_v, maxk_v, candid_v, candkey_v,
             celk_v, celi_v, sca, sem_ia, sem_ib, sem_oa, sem_ob):
    nc = 2
    wid = lax.axis_index("s") * nc + lax.axis_index("c")
    iota16 = lax.iota(jnp.int32, 16)
    ninf16 = jnp.full((16,), -jnp.inf, jnp.float32)
    negk16 = _splat(_INT_MIN)
    bufs = (bufa_v, bufb_v)
    sem_in = (sem_ia, sem_ib)
    sem_out = (sem_oa, sem_ob)

    def xmax_scalar(v16):
        ks, _ = plsc.sort_key_val(v16, v16)
        return lax.squeeze(lax.slice(ks, (15,), (16,)), dimensions=(0,))

    def bsearch(lo0, hi0, count_ge, target, iters):
        sca[0] = lo0
        sca[1] = hi0

        def step(_i, _c):
            @pl.when(sca[1] - sca[0] > 1)
            def _():
                lo = sca[0]
                hi = sca[1]
                mid = _mid_safe(lo, hi)
                ge = count_ge(mid) >= target
                sca[0] = jnp.where(ge, mid, lo)
                sca[1] = jnp.where(ge, hi, mid)
            return 0

        lax.fori_loop(0, iters, step, 0)
        return sca[0]

    def bsearch_window(lo0, hi0, count_ge, target, cap, iters):
        sca[0] = lo0
        sca[1] = hi0
        sca[2] = jnp.int32(0)

        def step(_i, _c):
            @pl.when((sca[2] == 0) & (sca[1] - sca[0] > 1))
            def _():
                lo = sca[0]
                hi = sca[1]
                mid = _mid_safe(lo, hi)
                c = count_ge(mid)
                ge = c >= target
                sca[0] = jnp.where(ge, mid, lo)
                sca[1] = jnp.where(ge, hi, mid)
                sca[2] = jnp.where(ge & (c <= cap), 1, 0)
            return 0

        lax.fori_loop(0, iters, step, 0)
        return sca[0]

    def chunk_stats(buf):
        def ga(g, carry):
            gmx, gmn = carry
            base = g * 512
            acc = buf[pl.ds(base, 16)]
            for i in range(1, 32):
                acc = jnp.maximum(acc, buf[pl.ds(base + i * 16, 16)])
            k = _key16(acc)
            maxk_v[pl.ds(g * 16, 16)] = k
            return jnp.maximum(gmx, k), jnp.minimum(gmn, k)

        gmx16, gmn16 = lax.fori_loop(0, _GROUPS, ga,
                                     (negk16, _splat(_INT_MAX)))
        gmax = xmax_scalar(gmx16)
        gminn = -xmax_scalar(-gmn16 - 1) - 1
        return gmax, gminn

    def cnt_maxk_ge(t):
        ts = _splat(t)
        acc = jnp.zeros((16,), jnp.int32)
        for p in range(_GROUPS):
            k = maxk_v[pl.ds(p * 16, 16)]
            acc = acc + plsc.all_reduce_population_count(k >= ts)
        return _lane0(acc)

    def compact_candidates(m50):
        m50s = _splat(m50)

        def cb(p, off):
            k = maxk_v[pl.ds(p * 16, 16)]
            m = k >= m50s
            ids = _splat(p * 16) + iota16
            plsc.store_compressed(candid_v.at[pl.ds(off, 16)], ids, mask=m)
            return off + _lane0(plsc.all_reduce_population_count(m))

        return lax.fori_loop(0, _GROUPS, cb, jnp.int32(0))

    npacks = _FAST_CHUNKS // 16

    def cand_pack(ci, ncs):
        cc = candid_v[pl.ds(ci * 16, 16)]
        valid = (_splat(ci * 16) + iota16) < ncs
        ccs = jnp.where(valid, cc, 0)
        base = ((ccs >> 4) << 9) + (ccs & 15)
        return base, valid

    def fast_row(j, buf, r):
        gmax, gminn = chunk_stats(buf)
        m50 = bsearch_window(gminn, gmax + 1, cnt_maxk_ge, _TOPK,
                             _FAST_CHUNKS, 32)
        ncand = compact_candidates(m50)
        ncs = _splat(ncand)

        @pl.when(ncand <= _FAST_CHUNKS)
        def _():
            def gc(ci, _):
                base, valid = cand_pack(ci, ncs)
                for i in range(32):
                    k = _key16(plsc.load_gather(buf, [base + i * 16]))
                    k = jnp.where(valid, k, negk16)
                    candkey_v[pl.ds((ci * 32 + i) * 16, 16)] = k
                return 0

            lax.fori_loop(0, npacks, gc, jnp.int32(0))

            def cnt_ge(t):
                ts = _splat(t)
                acc = jnp.zeros((16,), jnp.int32)
                for s in range(npacks * 32):
                    k = candkey_v[pl.ds(s * 16, 16)]
                    acc = acc + plsc.all_reduce_population_count(k >= ts)
                return _lane0(acc)

            p2 = bsearch_window(m50, gmax + 1, cnt_ge, _TOPK, 64, 32)
            p2s = _splat(p2)

            def ce(ci, off):
                base, _v = cand_pack(ci, ncs)
                o = off
                for i in range(32):
                    k = candkey_v[pl.ds((ci * 32 + i) * 16, 16)]
                    m = k >= p2s
                    plsc.store_compressed(celk_v.at[pl.ds(o, 16)], k, mask=m)
                    plsc.store_compressed(celi_v.at[pl.ds(o, 16)],
                                          base + i * 16, mask=m)
                    o = o + _lane0(plsc.all_reduce_population_count(m))
                return o

            c2 = lax.fori_loop(0, npacks, ce, jnp.int32(0))
            c2s = _splat(c2)

            @pl.when(c2 <= 64)
            def _():
                def cnt2(t):
                    ts2 = _splat(t)
                    acc = jnp.zeros((16,), jnp.int32)
                    for s in range(4):
                        k = celk_v[pl.ds(s * 16, 16)]
                        k = jnp.where((_splat(s * 16) + iota16) < c2s, k,
                                      negk16)
                        acc = acc + plsc.all_reduce_population_count(k >= ts2)
                    return _lane0(acc)

                tthr = bsearch(p2, gmax + 1, cnt2, _TOPK, 32)
                exact = cnt2(tthr) == _TOPK
                sca[3 + j] = jnp.where(exact, 0, 1)

                @pl.when(exact)
                def _():
                    ts = _splat(tthr)
                    for s in range(4):
                        k = celk_v[pl.ds(s * 16, 16)]
                        k = jnp.where((_splat(s * 16) + iota16) < c2s, k,
                                      negk16)
                        ix = celi_v[pl.ds(s * 16, 16)]
                        plsc.store_scatter(buf, [ix], ninf16, mask=k >= ts)

            @pl.when(c2 > 64)
            def _():
                sca[3 + j] = jnp.int32(1)

        @pl.when(ncand > _FAST_CHUNKS)
        def _():
            sca[3 + j] = jnp.int32(1)

    def cnt_row_mode(mode, t, bound):
        ts = _splat(t)
        bs = _splat(bound)
        m0 = _splat(mode) == 0
        m1 = _splat(mode) == 1

        def body(g, acc):
            a = acc
            base = g * 512
            for i in range(32):
                k = _key16(bufa_v[pl.ds(base + i * 16, 16)])
                idx = _splat(base + i * 16) + iota16
                m = jnp.where(m0, k >= ts,
                              jnp.where(m1, k > ts,
                                        (k == ts) & (idx <= bs)))
                a = a + plsc.all_reduce_population_count(m)
            return a

        return _lane0(lax.fori_loop(0, _GROUPS, body,
                                    jnp.zeros((16,), jnp.int32)))

    def cleanup_row(r):
        pltpu.sync_copy(in_hbm.at[r], bufa_v)
        tthr = bsearch(jnp.int32(_INT_MIN), jnp.int32(_INT_MAX),
                       lambda t: cnt_row_mode(jnp.int32(0), t, jnp.int32(0)),
                       _TOPK, 32)
        cg = cnt_row_mode(jnp.int32(1), tthr, jnp.int32(0))
        rem = _TOPK - cg

        sca[0] = jnp.int32(-1)
        sca[1] = jnp.int32(_N - 1)

        def istep(_i, _c):
            @pl.when(sca[1] - sca[0] > 1)
            def _():
                lo = sca[0]
                hi = sca[1]
                mid = lo + ((hi - lo) >> 1)
                ge = cnt_row_mode(jnp.int32(2), tthr, mid) >= rem
                sca[0] = jnp.where(ge, lo, mid)
                sca[1] = jnp.where(ge, mid, hi)
            return 0

        lax.fori_loop(0, 15, istep, 0)
        isel = sca[1]

        ts = _splat(tthr)
        iss = _splat(isel)

        def rw(g, _):
            base = g * 512
            for i in range(32):
                sl = pl.ds(base + i * 16, 16)
                v = bufa_v[sl]
                k = _key16(v)
                idx = _splat(base + i * 16) + iota16
                m = (k > ts) | ((k == ts) & (idx <= iss))
                bufa_v[sl] = jnp.where(m, ninf16, v)
            return 0

        lax.fori_loop(0, _GROUPS, rw, 0)
        pltpu.sync_copy(bufa_v, out_hbm.at[r])

    in_desc = [None, None]
    out_desc = [None, None]
    in_desc[0] = pltpu.async_copy(in_hbm.at[wid * 4], bufs[0], sem_in[0])
    for j in range(4):
        p = j & 1
        r = wid * 4 + j
        in_desc[p].wait()
        if j >= 1:
            out_desc[1 - p].wait()
        if j < 3:
            in_desc[1 - p] = pltpu.async_copy(in_hbm.at[r + 1], bufs[1 - p],
                                              sem_in[1 - p])
        fast_row(j, bufs[p], r)
        out_desc[p] = pltpu.async_copy(bufs[p], out_hbm.at[r], sem_out[p])
    out_desc[1].wait()

    def cl(j, _):
        @pl.when(sca[3 + j] == 1)
        def _():
            cleanup_row(wid * 4 + j)
        return 0

    lax.fori_loop(0, 4, cl, 0)


def kernel(scores):
    b, n = scores.shape
    mesh = plsc.VectorSubcoreMesh(core_axis_name="c", subcore_axis_name="s")
    return pl.kernel(
        _sc_body,
        out_type=jax.ShapeDtypeStruct((b, n), jnp.float32),
        mesh=mesh,
        compiler_params=pltpu.CompilerParams(needs_layout_passes=False),
        scratch_types=[
            pltpu.VMEM((_N,), jnp.float32),
            pltpu.VMEM((_N,), jnp.float32),
            pltpu.VMEM((_NCHUNK,), jnp.int32),
            pltpu.VMEM((_NCHUNK + 16,), jnp.int32),
            pltpu.VMEM((_FAST_CHUNKS * 32,), jnp.int32),
            pltpu.VMEM((_FAST_CHUNKS * 32 + 16,), jnp.int32),
            pltpu.VMEM((_FAST_CHUNKS * 32 + 16,), jnp.int32),
            pltpu.SMEM((8,), jnp.int32),
            pltpu.SemaphoreType.DMA,
            pltpu.SemaphoreType.DMA,
            pltpu.SemaphoreType.DMA,
            pltpu.SemaphoreType.DMA,
        ],
    )(scores)


# speedup vs baseline: 8.6157x; 1.0009x over previous
import jax
import jax.numpy as jnp
from jax import lax
from jax.experimental import pallas as pl
from jax.experimental.pallas import tpu as pltpu
from jax.experimental.pallas import tpu_sc as plsc

_TOPK = 50
_N = 32768
_GROUPS = 64
_NCHUNK = _GROUPS * 16
_FAST_CHUNKS = 64
_INT_MIN = -2147483648
_INT_MAX = 2147483647


def _key16(v):
    u = lax.bitcast_convert_type(v, jnp.int32)
    return jnp.where(u >= 0, u, jnp.int32(_INT_MIN) - u)


def _splat(x):
    return jnp.full((16,), x, jnp.int32)


def _mid_safe(lo, hi):
    return (lo >> 1) + (hi >> 1) + (lo & hi & 1)


def _lane0(x):
    return lax.squeeze(lax.slice(x, (0,), (1,)), dimensions=(0,))


def _sc_body(in_hbm, out_hbm, bufa_v, bufb_v, maxk_v, candid_v, candkey_v,
             celk_v, celi_v, sca, sem_ia, sem_ib, sem_oa, sem_ob):
    nc = 2
    wid = lax.axis_index("s") * nc + lax.axis_index("c")
    iota16 = lax.iota(jnp.int32, 16)
    ninf16 = jnp.full((16,), -jnp.inf, jnp.float32)
    negk16 = _splat(_INT_MIN)
    bufs = (bufa_v, bufb_v)
    sem_in = (sem_ia, sem_ib)
    sem_out = (sem_oa, sem_ob)

    def xmax_scalar(v16):
        ks, _ = plsc.sort_key_val(v16, v16)
        return lax.squeeze(lax.slice(ks, (15,), (16,)), dimensions=(0,))

    def bsearch(lo0, hi0, count_ge, target, iters):
        sca[0] = lo0
        sca[1] = hi0

        def step(_i, _c):
            @pl.when(sca[1] - sca[0] > 1)
            def _():
                lo = sca[0]
                hi = sca[1]
                mid = _mid_safe(lo, hi)
                ge = count_ge(mid) >= target
                sca[0] = jnp.where(ge, mid, lo)
                sca[1] = jnp.where(ge, hi, mid)
            return 0

        lax.fori_loop(0, iters, step, 0)
        return sca[0]

    def bsearch_window(lo0, hi0, count_ge, target, cap, iters):
        sca[0] = lo0
        sca[1] = hi0
        sca[2] = jnp.int32(0)

        def step(_i, _c):
            @pl.when((sca[2] == 0) & (sca[1] - sca[0] > 1))
            def _():
                lo = sca[0]
                hi = sca[1]
                mid = _mid_safe(lo, hi)
                c = count_ge(mid)
                ge = c >= target
                sca[0] = jnp.where(ge, mid, lo)
                sca[1] = jnp.where(ge, hi, mid)
                sca[2] = jnp.where(ge & (c <= cap), 1, 0)
            return 0

        lax.fori_loop(0, iters, step, 0)
        return sca[0]

    def chunk_stats(buf):
        def ga(g, carry):
            gmx, gmn = carry
            base = g * 512
            acc = buf[pl.ds(base, 16)]
            for i in range(1, 32):
                acc = jnp.maximum(acc, buf[pl.ds(base + i * 16, 16)])
            k = _key16(acc)
            maxk_v[pl.ds(g * 16, 16)] = k
            return jnp.maximum(gmx, k), jnp.minimum(gmn, k)

        gmx16, gmn16 = lax.fori_loop(0, _GROUPS, ga,
                                     (negk16, _splat(_INT_MAX)))
        gmax = xmax_scalar(gmx16)
        gminn = -xmax_scalar(-gmn16 - 1) - 1
        return gmax, gminn

    def cnt_maxk_ge(t):
        ts = _splat(t)
        acc = jnp.zeros((16,), jnp.int32)
        for p in range(_GROUPS):
            k = maxk_v[pl.ds(p * 16, 16)]
            acc = acc + plsc.all_reduce_population_count(k >= ts)
        return _lane0(acc)

    def compact_candidates(m50):
        m50s = _splat(m50)

        def cb(p, off):
            k = maxk_v[pl.ds(p * 16, 16)]
            m = k >= m50s
            ids = _splat(p * 16) + iota16
            plsc.store_compressed(candid_v.at[pl.ds(off, 16)], ids, mask=m)
            return off + _lane0(plsc.all_reduce_population_count(m))

        return lax.fori_loop(0, _GROUPS, cb, jnp.int32(0))

    npacks = _FAST_CHUNKS // 16

    def cand_pack(ci, ncs):
        cc = candid_v[pl.ds(ci * 16, 16)]
        valid = (_splat(ci * 16) + iota16) < ncs
        ccs = jnp.where(valid, cc, 0)
        base = ((ccs >> 4) << 9) + (ccs & 15)
        return base, valid

    def fast_row(j, buf, r):
        gmax, gminn = chunk_stats(buf)
        m50 = bsearch_window(gminn, gmax + 1, cnt_maxk_ge, _TOPK,
                             _FAST_CHUNKS, 32)
        ncand = compact_candidates(m50)
        ncs = _splat(ncand)

        @pl.when(ncand <= _FAST_CHUNKS)
        def _():
            def gc(ci, _):
                base, valid = cand_pack(ci, ncs)
                for i in range(32):
                    k = _key16(plsc.load_gather(buf, [base + i * 16]))
                    k = jnp.where(valid, k, negk16)
                    candkey_v[pl.ds((ci * 32 + i) * 16, 16)] = k
                return 0

            lax.fori_loop(0, npacks, gc, jnp.int32(0))

            def cnt_ge(t):
                ts = _splat(t)
                acc = jnp.zeros((16,), jnp.int32)
                for s in range(npacks * 32):
                    k = candkey_v[pl.ds(s * 16, 16)]
                    acc = acc + plsc.all_reduce_population_count(k >= ts)
                return _lane0(acc)

            p2 = bsearch_window(m50, gmax + 1, cnt_ge, _TOPK, 64, 32)
            p2s = _splat(p2)

            def ce(ci, off):
                base, _v = cand_pack(ci, ncs)
                o = off
                for i in range(32):
                    k = candkey_v[pl.ds((ci * 32 + i) * 16, 16)]
                    m = k >= p2s
                    plsc.store_compressed(celk_v.at[pl.ds(o, 16)], k, mask=m)
                    plsc.store_compressed(celi_v.at[pl.ds(o, 16)],
                                          base + i * 16, mask=m)
                    o = o + _lane0(plsc.all_reduce_population_count(m))
                return o

            c2 = lax.fori_loop(0, npacks, ce, jnp.int32(0))
            c2s = _splat(c2)

            @pl.when(c2 <= 64)
            def _():
                def cnt2(t):
                    ts2 = _splat(t)
                    acc = jnp.zeros((16,), jnp.int32)
                    for s in range(4):
                        k = celk_v[pl.ds(s * 16, 16)]
                        k = jnp.where((_splat(s * 16) + iota16) < c2s, k,
                                      negk16)
                        acc = acc + plsc.all_reduce_population_count(k >= ts2)
                    return _lane0(acc)

                tthr = bsearch(p2, gmax + 1, cnt2, _TOPK, 32)
                exact = cnt2(tthr) == _TOPK
                sca[3] = sca[3] | jnp.where(exact, 0, 1 << j)

                @pl.when(exact)
                def _():
                    ts = _splat(tthr)
                    for s in range(4):
                        k = celk_v[pl.ds(s * 16, 16)]
                        k = jnp.where((_splat(s * 16) + iota16) < c2s, k,
                                      negk16)
                        ix = celi_v[pl.ds(s * 16, 16)]
                        plsc.store_scatter(buf, [ix], ninf16, mask=k >= ts)

            @pl.when(c2 > 64)
            def _():
                sca[3] = sca[3] | (1 << j)

        @pl.when(ncand > _FAST_CHUNKS)
        def _():
            sca[3] = sca[3] | (1 << j)

    def cnt_row_mode(mode, t, bound):
        ts = _splat(t)
        bs = _splat(bound)
        m0 = _splat(mode) == 0
        m1 = _splat(mode) == 1

        def body(g, acc):
            a = acc
            base = g * 512
            for i in range(32):
                k = _key16(bufa_v[pl.ds(base + i * 16, 16)])
                idx = _splat(base + i * 16) + iota16
                m = ((m0 & (k >= ts)) | (m1 & (k > ts))
                     | ((~m0) & (~m1) & (k == ts) & (idx <= bs)))
                a = a + plsc.all_reduce_population_count(m)
            return a

        return _lane0(lax.fori_loop(0, _GROUPS, body,
                                    jnp.zeros((16,), jnp.int32)))

    def cleanup_row(r):
        pltpu.sync_copy(in_hbm.at[r], bufa_v)
        tthr = bsearch(jnp.int32(_INT_MIN), jnp.int32(_INT_MAX),
                       lambda t: cnt_row_mode(jnp.int32(0), t, jnp.int32(0)),
                       _TOPK, 32)
        cg = cnt_row_mode(jnp.int32(1), tthr, jnp.int32(0))
        rem = _TOPK - cg

        sca[0] = jnp.int32(-1)
        sca[1] = jnp.int32(_N - 1)

        def istep(_i, _c):
            @pl.when(sca[1] - sca[0] > 1)
            def _():
                lo = sca[0]
                hi = sca[1]
                mid = lo + ((hi - lo) >> 1)
                ge = cnt_row_mode(jnp.int32(2), tthr, mid) >= rem
                sca[0] = jnp.where(ge, lo, mid)
                sca[1] = jnp.where(ge, mid, hi)
            return 0

        lax.fori_loop(0, 15, istep, 0)
        isel = sca[1]

        ts = _splat(tthr)
        iss = _splat(isel)

        def rw(g, _):
            base = g * 512
            for i in range(32):
                sl = pl.ds(base + i * 16, 16)
                v = bufa_v[sl]
                k = _key16(v)
                idx = _splat(base + i * 16) + iota16
                m = (k > ts) | ((k == ts) & (idx <= iss))
                bufa_v[sl] = jnp.where(m, ninf16, v)
            return 0

        lax.fori_loop(0, _GROUPS, rw, 0)
        pltpu.sync_copy(bufa_v, out_hbm.at[r])

    sca[3] = jnp.int32(0)
    in_desc = [None, None]
    out_desc = [None, None]
    in_desc[0] = pltpu.async_copy(in_hbm.at[wid * 4], bufs[0], sem_in[0])
    for j in range(4):
        p = j & 1
        r = wid * 4 + j
        in_desc[p].wait()
        if j >= 1:
            out_desc[1 - p].wait()
        if j < 3:
            in_desc[1 - p] = pltpu.async_copy(in_hbm.at[r + 1], bufs[1 - p],
                                              sem_in[1 - p])
        fast_row(j, bufs[p], r)
        out_desc[p] = pltpu.async_copy(bufs[p], out_hbm.at[r], sem_out[p])
    out_desc[1].wait()

    def cl(j, _):
        @pl.when((lax.shift_right_logical(sca[3], j) & 1) == 1)
        def _():
            cleanup_row(wid * 4 + j)
        return 0

    lax.fori_loop(0, 4, cl, 0)


def kernel(scores):
    b, n = scores.shape
    mesh = plsc.VectorSubcoreMesh(core_axis_name="c", subcore_axis_name="s")
    return pl.kernel(
        _sc_body,
        out_type=jax.ShapeDtypeStruct((b, n), jnp.float32),
        mesh=mesh,
        compiler_params=pltpu.CompilerParams(needs_layout_passes=False),
        scratch_types=[
            pltpu.VMEM((_N,), jnp.float32),
            pltpu.VMEM((_N,), jnp.float32),
            pltpu.VMEM((_NCHUNK,), jnp.int32),
            pltpu.VMEM((_NCHUNK + 16,), jnp.int32),
            pltpu.VMEM((_FAST_CHUNKS * 32,), jnp.int32),
            pltpu.VMEM((_FAST_CHUNKS * 32 + 16,), jnp.int32),
            pltpu.VMEM((_FAST_CHUNKS * 32 + 16,), jnp.int32),
            pltpu.SMEM((8,), jnp.int32),
            pltpu.SemaphoreType.DMA,
            pltpu.SemaphoreType.DMA,
            pltpu.SemaphoreType.DMA,
            pltpu.SemaphoreType.DMA,
        ],
    )(scores)
